# baseline (speedup 1.0000x reference)
import functools

import jax
import jax.numpy as jnp
from jax import lax
from jax.experimental import pallas as pl
from jax.experimental.pallas import tpu as pltpu
from jax.experimental.pallas import tpu_sc as plsc

_N = 10000
_E = 320000
_D = 128
_G = 64

_NC = 2
_NS = 16
_NW = _NC * _NS
_CHUNK = 256
_CT = 40
_EPT = _CT * _CHUNK
_EPAD = _NW * _EPT
_NACC = 10112
_ZROWS = _NACC // _NS
_OROWS = _NACC // _NS

_BN = 1000
_NBLK = _N // _BN


@functools.cache
def _make_sc_agg():
    mesh = plsc.VectorSubcoreMesh(core_axis_name="c", subcore_axis_name="s")

    @functools.partial(
        pl.kernel,
        mesh=mesh,
        out_type=jax.ShapeDtypeStruct((_NC, _NACC, _D), jnp.float32),
        scratch_types=[
            pltpu.VMEM((_EPT // 128, 128), jnp.int32),
            pltpu.VMEM((_CHUNK,), jnp.int32),
            pltpu.VMEM((_CHUNK,), jnp.int32),
            pltpu.VMEM((_CHUNK,), jnp.int32),
            pltpu.VMEM((_CHUNK,), jnp.int32),
            pltpu.VMEM((_CHUNK, _D), jnp.float32),
            pltpu.VMEM_SHARED((_NACC, _D), jnp.float32),
            pltpu.SemaphoreType.DMA,
        ],
    )
    def agg(feat_hbm, edge_hbm, zeros_hbm, out_hbm,
            pk_v, src0_v, src1_v, dst0_v, dst1_v, rows_v, acc_sh, gsem):
        c = lax.axis_index("c")
        s = lax.axis_index("s")
        wid = c * _NS + s
        srcs = (src0_v, src1_v)
        dsts = (dst0_v, dst1_v)

        def unpack(chunk, buf):
            for q in range(_CHUNK // 128):
                for k in range(8):
                    p = pk_v[chunk * (_CHUNK // 128) + q, pl.ds(k * 16, 16)]
                    col = pl.ds(q * 128 + k * 16, 16)
                    srcs[buf][col] = lax.bitwise_and(p, 0xFFFF)
                    dsts[buf][col] = lax.shift_right_logical(p, 16)

        def gather(buf):
            return pltpu.make_async_copy(
                feat_hbm.at[srcs[buf]], rows_v, gsem)

        pltpu.sync_copy(zeros_hbm, acc_sh.at[pl.ds(s * _ZROWS, _ZROWS)])
        pltpu.sync_copy(edge_hbm.at[wid], pk_v)
        unpack(jnp.int32(0), 0)
        gather(0).start()
        plsc.subcore_barrier()

        def group(g, carry):
            for par in range(2):
                j = g * 2 + par
                @pl.when(j + 1 < _CT)
                def _prep_next():
                    unpack(j + 1, 1 - par)

                gather(par).wait()
                pltpu.sync_copy(rows_v, acc_sh.at[dsts[par]], add=True)

                @pl.when(j + 1 < _CT)
                def _gather_next():
                    gather(1 - par).start()
            return carry

        lax.fori_loop(0, _CT // 2, group, 0, unroll=False)

        plsc.subcore_barrier()
        pltpu.sync_copy(acc_sh.at[pl.ds(s * _OROWS, _OROWS)],
                        out_hbm.at[c, pl.ds(s * _OROWS, _OROWS)])

    return agg


def _sc_agg(feat, packed, zrows):
    return _make_sc_agg()(feat, packed, zrows)


def _mlp_body(x_ref, agg_ref, wa_ref, ba_ref, wb_ref, bb_ref, out_ref):
    h = x_ref[...] + agg_ref[0] + agg_ref[1]
    h = jnp.dot(h, wa_ref[...], preferred_element_type=jnp.float32) + ba_ref[...]
    h = jnp.maximum(h, 0.0)
    h = jnp.dot(h, wb_ref[...], preferred_element_type=jnp.float32) + bb_ref[...]
    out_ref[...] = jnp.maximum(h, 0.0)


def _tc_mlp1(x, agg, wa, ba, wb, bb):
    blk = lambda i: (i, 0)
    full = lambda i: (0, 0)
    return pl.pallas_call(
        _mlp_body,
        grid=(_NBLK,),
        in_specs=[
            pl.BlockSpec((_BN, _D), blk),
            pl.BlockSpec((_NC, _BN, _D), lambda i: (0, i, 0)),
            pl.BlockSpec((_D, _D), full),
            pl.BlockSpec((1, _D), full),
            pl.BlockSpec((_D, _D), full),
            pl.BlockSpec((1, _D), full),
        ],
        out_specs=pl.BlockSpec((_BN, _D), blk),
        out_shape=jax.ShapeDtypeStruct((_N, _D), jnp.float32),
    )(x, agg, wa, ba, wb, bb)


def _mlp_pool_body(batch_ref, x_ref, agg_ref, wa_ref, ba_ref, wb_ref,
                   bb_ref, out_ref, pool_ref):
    i = pl.program_id(0)
    h = x_ref[...] + agg_ref[0] + agg_ref[1]
    h = jnp.dot(h, wa_ref[...], preferred_element_type=jnp.float32) + ba_ref[...]
    h = jnp.maximum(h, 0.0)
    h = jnp.dot(h, wb_ref[...], preferred_element_type=jnp.float32) + bb_ref[...]
    out_ref[...] = h
    b = batch_ref[0, 0, :]
    onehot = (b[:, None] == lax.broadcasted_iota(jnp.int32, (_BN, _G), 1))
    contrib = lax.dot_general(onehot.astype(jnp.float32), h,
                              (((0,), (0,)), ((), ())),
                              preferred_element_type=jnp.float32)

    @pl.when(i == 0)
    def _init():
        pool_ref[...] = jnp.zeros_like(pool_ref)

    pool_ref[...] += contrib


def _tc_mlp2_pool(batch3, x, agg, wa, ba, wb, bb):
    blk = lambda i: (i, 0)
    full = lambda i: (0, 0)
    return pl.pallas_call(
        _mlp_pool_body,
        grid=(_NBLK,),
        in_specs=[
            pl.BlockSpec((1, 1, _BN), lambda i: (i, 0, 0)),
            pl.BlockSpec((_BN, _D), blk),
            pl.BlockSpec((_NC, _BN, _D), lambda i: (0, i, 0)),
            pl.BlockSpec((_D, _D), full),
            pl.BlockSpec((1, _D), full),
            pl.BlockSpec((_D, _D), full),
            pl.BlockSpec((1, _D), full),
        ],
        out_specs=[
            pl.BlockSpec((_BN, _D), blk),
            pl.BlockSpec((_G, _D), full),
        ],
        out_shape=[
            jax.ShapeDtypeStruct((_N, _D), jnp.float32),
            jax.ShapeDtypeStruct((_G, _D), jnp.float32),
        ],
    )(batch3, x, agg, wa, ba, wb, bb)


def kernel(x, edge_index, batch, W1a, b1a, W1b, b1b, W2a, b2a, W2b, b2b):
    src = edge_index[0]
    dst = edge_index[1]
    pad = _EPAD - _E
    srcp = jnp.concatenate([src, jnp.zeros((pad,), jnp.int32)])
    pad_dst = _N + (jnp.arange(pad, dtype=jnp.int32) % (_NACC - _N))
    dstp = jnp.concatenate([dst, pad_dst])
    packed = jnp.bitwise_or(srcp, jnp.left_shift(dstp, 16))
    packed = packed.reshape(_NW, _EPT // 128, 128)
    zrows = jnp.zeros((_ZROWS, _D), jnp.float32)

    ba1 = b1a.reshape(1, _D)
    bb1 = b1b.reshape(1, _D)
    ba2 = b2a.reshape(1, _D)
    bb2 = b2b.reshape(1, _D)

    agg1 = _sc_agg(x, packed, zrows)
    h1 = _tc_mlp1(x, agg1, W1a, ba1, W1b, bb1)
    agg2 = _sc_agg(h1, packed, zrows)
    batch3 = batch.reshape(_NBLK, 1, _BN)
    h2, pooled = _tc_mlp2_pool(batch3, h1, agg2, W2a, ba2, W2b, bb2)
    return (pooled, h2)


# speedup vs baseline: 3.0429x; 3.0429x over previous
import functools

import jax
import jax.numpy as jnp
from jax import lax
from jax.experimental import pallas as pl
from jax.experimental.pallas import tpu as pltpu
from jax.experimental.pallas import tpu_sc as plsc

_N = 10000
_E = 320000
_D = 128
_G = 64

_NC = 2
_NS = 16
_NW = _NC * _NS
_CHUNK = 256
_CT = 40
_EPT = _CT * _CHUNK
_EPAD = _NW * _EPT
_NACC = 10112
_ZROWS = _NACC // _NS
_OROWS = _NACC // _NS

_BN = 1000
_NBLK = _N // _BN


@functools.cache
def _make_sc_agg():
    mesh = plsc.VectorSubcoreMesh(core_axis_name="c", subcore_axis_name="s")

    @functools.partial(
        pl.kernel,
        mesh=mesh,
        out_type=jax.ShapeDtypeStruct((_NC, _NACC, _D), jnp.float32),
        scratch_types=[
            pltpu.VMEM((_EPT // 128, 128), jnp.int32),
            pltpu.VMEM((_CHUNK,), jnp.int32),
            pltpu.VMEM((_CHUNK,), jnp.int32),
            pltpu.VMEM((_CHUNK,), jnp.int32),
            pltpu.VMEM((_CHUNK,), jnp.int32),
            pltpu.VMEM((_CHUNK, _D), jnp.float32),
            pltpu.VMEM_SHARED((_NACC, _D), jnp.float32),
            pltpu.SemaphoreType.DMA,
        ],
    )
    def agg(feat_hbm, edge_hbm, zeros_hbm, out_hbm,
            pk_v, src0_v, src1_v, dst0_v, dst1_v, rows_v, acc_sh, gsem):
        c = lax.axis_index("c")
        s = lax.axis_index("s")
        wid = c * _NS + s
        srcs = (src0_v, src1_v)
        dsts = (dst0_v, dst1_v)

        def unpack(chunk, buf):
            for q in range(_CHUNK // 128):
                for k in range(8):
                    p = pk_v[chunk * (_CHUNK // 128) + q, pl.ds(k * 16, 16)]
                    col = pl.ds(q * 128 + k * 16, 16)
                    srcs[buf][col] = lax.bitwise_and(p, 0xFFFF)
                    dsts[buf][col] = lax.shift_right_logical(p, 16)

        def gather(buf):
            return pltpu.make_async_copy(
                feat_hbm.at[srcs[buf]], rows_v, gsem)

        pltpu.sync_copy(zeros_hbm, acc_sh.at[pl.ds(s * _ZROWS, _ZROWS)])
        pltpu.sync_copy(edge_hbm.at[wid], pk_v)
        unpack(jnp.int32(0), 0)
        gather(0).start()
        plsc.subcore_barrier()

        def group(g, carry):
            for par in range(2):
                j = g * 2 + par
                @pl.when(j + 1 < _CT)
                def _prep_next():
                    unpack(j + 1, 1 - par)

                gather(par).wait()
                pltpu.sync_copy(rows_v, acc_sh.at[dsts[par]], add=True)

                @pl.when(j + 1 < _CT)
                def _gather_next():
                    gather(1 - par).start()
            return carry

        lax.fori_loop(0, _CT // 2, group, 0, unroll=False)

        plsc.subcore_barrier()
        pltpu.sync_copy(acc_sh.at[pl.ds(s * _OROWS, _OROWS)],
                        out_hbm.at[c, pl.ds(s * _OROWS, _OROWS)])

    return agg


def _sc_agg(feat, packed, zrows):
    return _make_sc_agg()(feat, packed, zrows)


def _mlp_body(x_ref, agg_ref, wa_ref, ba_ref, wb_ref, bb_ref, out_ref):
    h = x_ref[...] + agg_ref[0] + agg_ref[1]
    h = jnp.dot(h, wa_ref[...], preferred_element_type=jnp.float32) + ba_ref[...]
    h = jnp.maximum(h, 0.0)
    h = jnp.dot(h, wb_ref[...], preferred_element_type=jnp.float32) + bb_ref[...]
    out_ref[...] = jnp.maximum(h, 0.0)


def _tc_mlp1(x, agg, wa, ba, wb, bb):
    blk = lambda i: (i, 0)
    full = lambda i: (0, 0)
    return pl.pallas_call(
        _mlp_body,
        grid=(_NBLK,),
        in_specs=[
            pl.BlockSpec((_BN, _D), blk),
            pl.BlockSpec((_NC, _BN, _D), lambda i: (0, i, 0)),
            pl.BlockSpec((_D, _D), full),
            pl.BlockSpec((1, _D), full),
            pl.BlockSpec((_D, _D), full),
            pl.BlockSpec((1, _D), full),
        ],
        out_specs=pl.BlockSpec((_BN, _D), blk),
        out_shape=jax.ShapeDtypeStruct((_N, _D), jnp.float32),
    )(x, agg, wa, ba, wb, bb)


def _mlp_pool_body(batch_ref, x_ref, agg_ref, wa_ref, ba_ref, wb_ref,
                   bb_ref, out_ref, pool_ref):
    i = pl.program_id(0)
    h = x_ref[...] + agg_ref[0] + agg_ref[1]
    h = jnp.dot(h, wa_ref[...], preferred_element_type=jnp.float32) + ba_ref[...]
    h = jnp.maximum(h, 0.0)
    h = jnp.dot(h, wb_ref[...], preferred_element_type=jnp.float32) + bb_ref[...]
    out_ref[...] = h
    b = batch_ref[0, 0, :]
    onehot = (b[:, None] == lax.broadcasted_iota(jnp.int32, (_BN, _G), 1))
    contrib = lax.dot_general(onehot.astype(jnp.float32), h,
                              (((0,), (0,)), ((), ())),
                              preferred_element_type=jnp.float32)

    @pl.when(i == 0)
    def _init():
        pool_ref[...] = jnp.zeros_like(pool_ref)

    pool_ref[...] += contrib


def _tc_mlp2_pool(batch3, x, agg, wa, ba, wb, bb):
    blk = lambda i: (i, 0)
    full = lambda i: (0, 0)
    return pl.pallas_call(
        _mlp_pool_body,
        grid=(_NBLK,),
        in_specs=[
            pl.BlockSpec((1, 1, _BN), lambda i: (i, 0, 0)),
            pl.BlockSpec((_BN, _D), blk),
            pl.BlockSpec((_NC, _BN, _D), lambda i: (0, i, 0)),
            pl.BlockSpec((_D, _D), full),
            pl.BlockSpec((1, _D), full),
            pl.BlockSpec((_D, _D), full),
            pl.BlockSpec((1, _D), full),
        ],
        out_specs=[
            pl.BlockSpec((_BN, _D), blk),
            pl.BlockSpec((_G, _D), full),
        ],
        out_shape=[
            jax.ShapeDtypeStruct((_N, _D), jnp.float32),
            jax.ShapeDtypeStruct((_G, _D), jnp.float32),
        ],
    )(batch3, x, agg, wa, ba, wb, bb)


def kernel(x, edge_index, batch, W1a, b1a, W1b, b1b, W2a, b2a, W2b, b2b):
    src = edge_index[0]
    dst = edge_index[1]
    pad = _EPAD - _E
    pad_src = jnp.arange(pad, dtype=jnp.int32) % _N
    pad_dst = _N + (jnp.arange(pad, dtype=jnp.int32) % (_NACC - _N))
    srcp = jnp.concatenate([src, pad_src])
    dstp = jnp.concatenate([dst, pad_dst])
    packed = jnp.bitwise_or(srcp, jnp.left_shift(dstp, 16))
    packed = packed.reshape(_NW, _EPT // 128, 128)
    zrows = jnp.zeros((_ZROWS, _D), jnp.float32)

    ba1 = b1a.reshape(1, _D)
    bb1 = b1b.reshape(1, _D)
    ba2 = b2a.reshape(1, _D)
    bb2 = b2b.reshape(1, _D)

    agg1 = _sc_agg(x, packed, zrows)
    h1 = _tc_mlp1(x, agg1, W1a, ba1, W1b, bb1)
    agg2 = _sc_agg(h1, packed, zrows)
    batch3 = batch.reshape(_NBLK, 1, _BN)
    h2, pooled = _tc_mlp2_pool(batch3, h1, agg2, W2a, ba2, W2b, bb2)
    return (pooled, h2)


# speedup vs baseline: 3.4271x; 1.1262x over previous
import functools

import jax
import jax.numpy as jnp
from jax import lax
from jax.experimental import pallas as pl
from jax.experimental.pallas import tpu as pltpu
from jax.experimental.pallas import tpu_sc as plsc

_N = 10000
_E = 320000
_D = 128
_G = 64

_NC = 2
_NS = 16
_NW = _NC * _NS
_CHUNK = 128
_CT = 80
_EPT = _CT * _CHUNK
_EPAD = _NW * _EPT
_NACC = 10112
_ZROWS = _NACC // _NS
_OROWS = _NACC // _NS

_BN = 1000
_NBLK = _N // _BN


@functools.cache
def _make_sc_agg():
    mesh = plsc.VectorSubcoreMesh(core_axis_name="c", subcore_axis_name="s")

    @functools.partial(
        pl.kernel,
        mesh=mesh,
        out_type=jax.ShapeDtypeStruct((_NC, _NACC, _D), jnp.float32),
        scratch_types=[
            pltpu.VMEM((_EPT // 128, 128), jnp.int32),
            pltpu.VMEM((_CHUNK,), jnp.int32),
            pltpu.VMEM((_CHUNK,), jnp.int32),
            pltpu.VMEM((_CHUNK,), jnp.int32),
            pltpu.VMEM((_CHUNK,), jnp.int32),
            pltpu.VMEM((_CHUNK, _D), jnp.float32),
            pltpu.VMEM((_CHUNK, _D), jnp.float32),
            pltpu.VMEM_SHARED((_NACC, _D), jnp.float32),
            pltpu.SemaphoreType.DMA,
            pltpu.SemaphoreType.DMA,
        ],
    )
    def agg(feat_hbm, edge_hbm, zeros_hbm, out_hbm,
            pk_v, src0_v, src1_v, dst0_v, dst1_v, rows0_v, rows1_v,
            acc_sh, gsem, ssem):
        c = lax.axis_index("c")
        s = lax.axis_index("s")
        wid = c * _NS + s
        srcs = (src0_v, src1_v)
        dsts = (dst0_v, dst1_v)
        rows = (rows0_v, rows1_v)

        def unpack(chunk, buf):
            for q in range(_CHUNK // 128):
                for k in range(8):
                    p = pk_v[chunk * (_CHUNK // 128) + q, pl.ds(k * 16, 16)]
                    col = pl.ds(q * 128 + k * 16, 16)
                    srcs[buf][col] = lax.bitwise_and(p, 0xFFFF)
                    dsts[buf][col] = lax.shift_right_logical(p, 16)

        def gather(buf):
            return pltpu.make_async_copy(
                feat_hbm.at[srcs[buf]], rows[buf], gsem)

        def scatter(buf):
            return pltpu.make_async_copy(
                rows[buf], acc_sh.at[dsts[buf]], ssem)

        pltpu.sync_copy(zeros_hbm, acc_sh.at[pl.ds(s * _ZROWS, _ZROWS)])
        pltpu.sync_copy(edge_hbm.at[wid], pk_v)
        unpack(jnp.int32(0), 0)
        gather(0).start()
        plsc.subcore_barrier()

        def group(g, carry):
            for par in range(2):
                j = g * 2 + par
                @pl.when(j >= 1)
                def _wait_prev_scatter():
                    scatter(1 - par).wait()

                @pl.when(j + 1 < _CT)
                def _prep_next():
                    unpack(j + 1, 1 - par)

                gather(par).wait()

                @pl.when(j + 1 < _CT)
                def _gather_next():
                    gather(1 - par).start()

                pltpu.async_copy(
                    rows[par], acc_sh.at[dsts[par]], ssem, add=True)
            return carry

        lax.fori_loop(0, _CT // 2, group, 0, unroll=False)
        scatter((_CT - 1) % 2).wait()

        plsc.subcore_barrier()
        pltpu.sync_copy(acc_sh.at[pl.ds(s * _OROWS, _OROWS)],
                        out_hbm.at[c, pl.ds(s * _OROWS, _OROWS)])

    return agg


def _sc_agg(feat, packed, zrows):
    return _make_sc_agg()(feat, packed, zrows)


def _mlp_body(x_ref, agg_ref, wa_ref, ba_ref, wb_ref, bb_ref, out_ref):
    h = x_ref[...] + agg_ref[0] + agg_ref[1]
    h = jnp.dot(h, wa_ref[...], preferred_element_type=jnp.float32) + ba_ref[...]
    h = jnp.maximum(h, 0.0)
    h = jnp.dot(h, wb_ref[...], preferred_element_type=jnp.float32) + bb_ref[...]
    out_ref[...] = jnp.maximum(h, 0.0)


def _tc_mlp1(x, agg, wa, ba, wb, bb):
    blk = lambda i: (i, 0)
    full = lambda i: (0, 0)
    return pl.pallas_call(
        _mlp_body,
        grid=(_NBLK,),
        in_specs=[
            pl.BlockSpec((_BN, _D), blk),
            pl.BlockSpec((_NC, _BN, _D), lambda i: (0, i, 0)),
            pl.BlockSpec((_D, _D), full),
            pl.BlockSpec((1, _D), full),
            pl.BlockSpec((_D, _D), full),
            pl.BlockSpec((1, _D), full),
        ],
        out_specs=pl.BlockSpec((_BN, _D), blk),
        out_shape=jax.ShapeDtypeStruct((_N, _D), jnp.float32),
    )(x, agg, wa, ba, wb, bb)


def _mlp_pool_body(batch_ref, x_ref, agg_ref, wa_ref, ba_ref, wb_ref,
                   bb_ref, out_ref, pool_ref):
    i = pl.program_id(0)
    h = x_ref[...] + agg_ref[0] + agg_ref[1]
    h = jnp.dot(h, wa_ref[...], preferred_element_type=jnp.float32) + ba_ref[...]
    h = jnp.maximum(h, 0.0)
    h = jnp.dot(h, wb_ref[...], preferred_element_type=jnp.float32) + bb_ref[...]
    out_ref[...] = h
    b = batch_ref[0, 0, :]
    onehot = (b[:, None] == lax.broadcasted_iota(jnp.int32, (_BN, _G), 1))
    contrib = lax.dot_general(onehot.astype(jnp.float32), h,
                              (((0,), (0,)), ((), ())),
                              preferred_element_type=jnp.float32)

    @pl.when(i == 0)
    def _init():
        pool_ref[...] = jnp.zeros_like(pool_ref)

    pool_ref[...] += contrib


def _tc_mlp2_pool(batch3, x, agg, wa, ba, wb, bb):
    blk = lambda i: (i, 0)
    full = lambda i: (0, 0)
    return pl.pallas_call(
        _mlp_pool_body,
        grid=(_NBLK,),
        in_specs=[
            pl.BlockSpec((1, 1, _BN), lambda i: (i, 0, 0)),
            pl.BlockSpec((_BN, _D), blk),
            pl.BlockSpec((_NC, _BN, _D), lambda i: (0, i, 0)),
            pl.BlockSpec((_D, _D), full),
            pl.BlockSpec((1, _D), full),
            pl.BlockSpec((_D, _D), full),
            pl.BlockSpec((1, _D), full),
        ],
        out_specs=[
            pl.BlockSpec((_BN, _D), blk),
            pl.BlockSpec((_G, _D), full),
        ],
        out_shape=[
            jax.ShapeDtypeStruct((_N, _D), jnp.float32),
            jax.ShapeDtypeStruct((_G, _D), jnp.float32),
        ],
    )(batch3, x, agg, wa, ba, wb, bb)


def kernel(x, edge_index, batch, W1a, b1a, W1b, b1b, W2a, b2a, W2b, b2b):
    src = edge_index[0]
    dst = edge_index[1]
    pad = _EPAD - _E
    pad_src = jnp.arange(pad, dtype=jnp.int32) % _N
    pad_dst = _N + (jnp.arange(pad, dtype=jnp.int32) % (_NACC - _N))
    srcp = jnp.concatenate([src, pad_src])
    dstp = jnp.concatenate([dst, pad_dst])
    packed = jnp.bitwise_or(srcp, jnp.left_shift(dstp, 16))
    packed = packed.reshape(_NW, _EPT // 128, 128)
    zrows = jnp.zeros((_ZROWS, _D), jnp.float32)

    ba1 = b1a.reshape(1, _D)
    bb1 = b1b.reshape(1, _D)
    ba2 = b2a.reshape(1, _D)
    bb2 = b2b.reshape(1, _D)

    agg1 = _sc_agg(x, packed, zrows)
    h1 = _tc_mlp1(x, agg1, W1a, ba1, W1b, bb1)
    agg2 = _sc_agg(h1, packed, zrows)
    batch3 = batch.reshape(_NBLK, 1, _BN)
    h2, pooled = _tc_mlp2_pool(batch3, h1, agg2, W2a, ba2, W2b, bb2)
    return (pooled, h2)


# speedup vs baseline: 3.4735x; 1.0135x over previous
import functools

import jax
import jax.numpy as jnp
from jax import lax
from jax.experimental import pallas as pl
from jax.experimental.pallas import tpu as pltpu
from jax.experimental.pallas import tpu_sc as plsc

_N = 10000
_E = 320000
_D = 128
_G = 64

_NC = 2
_NS = 16
_NW = _NC * _NS
_CHUNK = 128
_CT = 80
_EPT = _CT * _CHUNK
_EPAD = _NW * _EPT
_NACC = 10112
_ZROWS = _NACC // _NS
_OROWS = _NACC // _NS

_BN = 2000
_NBLK = _N // _BN


@functools.cache
def _make_sc_agg():
    mesh = plsc.VectorSubcoreMesh(core_axis_name="c", subcore_axis_name="s")

    @functools.partial(
        pl.kernel,
        mesh=mesh,
        out_type=jax.ShapeDtypeStruct((_NC, _NACC, _D), jnp.float32),
        scratch_types=[
            pltpu.VMEM((_EPT // 128, 128), jnp.int32),
            pltpu.VMEM((_CHUNK,), jnp.int32),
            pltpu.VMEM((_CHUNK,), jnp.int32),
            pltpu.VMEM((_CHUNK,), jnp.int32),
            pltpu.VMEM((_CHUNK,), jnp.int32),
            pltpu.VMEM((_CHUNK, _D), jnp.float32),
            pltpu.VMEM((_CHUNK, _D), jnp.float32),
            pltpu.VMEM_SHARED((_NACC, _D), jnp.float32),
            pltpu.SemaphoreType.DMA,
            pltpu.SemaphoreType.DMA,
        ],
    )
    def agg(feat_hbm, edge_hbm, zeros_hbm, out_hbm,
            pk_v, src0_v, src1_v, dst0_v, dst1_v, rows0_v, rows1_v,
            acc_sh, gsem, ssem):
        c = lax.axis_index("c")
        s = lax.axis_index("s")
        wid = c * _NS + s
        srcs = (src0_v, src1_v)
        dsts = (dst0_v, dst1_v)
        rows = (rows0_v, rows1_v)

        def unpack(chunk, buf):
            for q in range(_CHUNK // 128):
                for k in range(8):
                    p = pk_v[chunk * (_CHUNK // 128) + q, pl.ds(k * 16, 16)]
                    col = pl.ds(q * 128 + k * 16, 16)
                    srcs[buf][col] = lax.bitwise_and(p, 0xFFFF)
                    dsts[buf][col] = lax.shift_right_logical(p, 16)

        def gather(buf):
            return pltpu.make_async_copy(
                feat_hbm.at[srcs[buf]], rows[buf], gsem)

        def scatter(buf):
            return pltpu.make_async_copy(
                rows[buf], acc_sh.at[dsts[buf]], ssem)

        pltpu.sync_copy(zeros_hbm, acc_sh.at[pl.ds(s * _ZROWS, _ZROWS)])
        pltpu.sync_copy(edge_hbm.at[wid], pk_v)
        unpack(jnp.int32(0), 0)
        gather(0).start()
        plsc.subcore_barrier()

        def group(g, carry):
            for par in range(2):
                j = g * 2 + par
                @pl.when(j >= 1)
                def _wait_prev_scatter():
                    scatter(1 - par).wait()

                @pl.when(j + 1 < _CT)
                def _prep_next():
                    unpack(j + 1, 1 - par)

                gather(par).wait()

                @pl.when(j + 1 < _CT)
                def _gather_next():
                    gather(1 - par).start()

                pltpu.async_copy(
                    rows[par], acc_sh.at[dsts[par]], ssem, add=True)
            return carry

        lax.fori_loop(0, _CT // 2, group, 0, unroll=False)
        scatter((_CT - 1) % 2).wait()

        plsc.subcore_barrier()
        pltpu.sync_copy(acc_sh.at[pl.ds(s * _OROWS, _OROWS)],
                        out_hbm.at[c, pl.ds(s * _OROWS, _OROWS)])

    return agg


def _sc_agg(feat, packed, zrows):
    return _make_sc_agg()(feat, packed, zrows)


def _mlp_body(x_ref, agg_ref, wa_ref, ba_ref, wb_ref, bb_ref, out_ref):
    h = x_ref[...] + agg_ref[0] + agg_ref[1]
    h = jnp.dot(h, wa_ref[...], preferred_element_type=jnp.float32) + ba_ref[...]
    h = jnp.maximum(h, 0.0)
    h = jnp.dot(h, wb_ref[...], preferred_element_type=jnp.float32) + bb_ref[...]
    out_ref[...] = jnp.maximum(h, 0.0)


def _tc_mlp1(x, agg, wa, ba, wb, bb):
    blk = lambda i: (i, 0)
    full = lambda i: (0, 0)
    return pl.pallas_call(
        _mlp_body,
        grid=(_NBLK,),
        in_specs=[
            pl.BlockSpec((_BN, _D), blk),
            pl.BlockSpec((_NC, _BN, _D), lambda i: (0, i, 0)),
            pl.BlockSpec((_D, _D), full),
            pl.BlockSpec((1, _D), full),
            pl.BlockSpec((_D, _D), full),
            pl.BlockSpec((1, _D), full),
        ],
        out_specs=pl.BlockSpec((_BN, _D), blk),
        out_shape=jax.ShapeDtypeStruct((_N, _D), jnp.float32),
    )(x, agg, wa, ba, wb, bb)


def _mlp_pool_body(batch_ref, x_ref, agg_ref, wa_ref, ba_ref, wb_ref,
                   bb_ref, out_ref, pool_ref):
    i = pl.program_id(0)
    h = x_ref[...] + agg_ref[0] + agg_ref[1]
    h = jnp.dot(h, wa_ref[...], preferred_element_type=jnp.float32) + ba_ref[...]
    h = jnp.maximum(h, 0.0)
    h = jnp.dot(h, wb_ref[...], preferred_element_type=jnp.float32) + bb_ref[...]
    out_ref[...] = h
    b = batch_ref[0, 0, :]
    onehot = (b[:, None] == lax.broadcasted_iota(jnp.int32, (_BN, _G), 1))
    contrib = lax.dot_general(onehot.astype(jnp.float32), h,
                              (((0,), (0,)), ((), ())),
                              preferred_element_type=jnp.float32)

    @pl.when(i == 0)
    def _init():
        pool_ref[...] = jnp.zeros_like(pool_ref)

    pool_ref[...] += contrib


def _tc_mlp2_pool(batch3, x, agg, wa, ba, wb, bb):
    blk = lambda i: (i, 0)
    full = lambda i: (0, 0)
    return pl.pallas_call(
        _mlp_pool_body,
        grid=(_NBLK,),
        in_specs=[
            pl.BlockSpec((1, 1, _BN), lambda i: (i, 0, 0)),
            pl.BlockSpec((_BN, _D), blk),
            pl.BlockSpec((_NC, _BN, _D), lambda i: (0, i, 0)),
            pl.BlockSpec((_D, _D), full),
            pl.BlockSpec((1, _D), full),
            pl.BlockSpec((_D, _D), full),
            pl.BlockSpec((1, _D), full),
        ],
        out_specs=[
            pl.BlockSpec((_BN, _D), blk),
            pl.BlockSpec((_G, _D), full),
        ],
        out_shape=[
            jax.ShapeDtypeStruct((_N, _D), jnp.float32),
            jax.ShapeDtypeStruct((_G, _D), jnp.float32),
        ],
    )(batch3, x, agg, wa, ba, wb, bb)


def kernel(x, edge_index, batch, W1a, b1a, W1b, b1b, W2a, b2a, W2b, b2b):
    src = edge_index[0]
    dst = edge_index[1]
    pad = _EPAD - _E
    pad_src = jnp.arange(pad, dtype=jnp.int32) % _N
    pad_dst = _N + (jnp.arange(pad, dtype=jnp.int32) % (_NACC - _N))
    srcp = jnp.concatenate([src, pad_src])
    dstp = jnp.concatenate([dst, pad_dst])
    packed = jnp.bitwise_or(srcp, jnp.left_shift(dstp, 16))
    packed = packed.reshape(_NW, _EPT // 128, 128)
    zrows = jnp.zeros((_ZROWS, _D), jnp.float32)

    ba1 = b1a.reshape(1, _D)
    bb1 = b1b.reshape(1, _D)
    ba2 = b2a.reshape(1, _D)
    bb2 = b2b.reshape(1, _D)

    agg1 = _sc_agg(x, packed, zrows)
    h1 = _tc_mlp1(x, agg1, W1a, ba1, W1b, bb1)
    agg2 = _sc_agg(h1, packed, zrows)
    batch3 = batch.reshape(_NBLK, 1, _BN)
    h2, pooled = _tc_mlp2_pool(batch3, h1, agg2, W2a, ba2, W2b, bb2)
    return (pooled, h2)


# speedup vs baseline: 4.0520x; 1.1666x over previous
import functools

import jax
import jax.numpy as jnp
from jax import lax
from jax.experimental import pallas as pl
from jax.experimental.pallas import tpu as pltpu
from jax.experimental.pallas import tpu_sc as plsc

_N = 10000
_E = 320000
_D = 128
_G = 64

_NC = 2
_NS = 16
_NW = _NC * _NS
_CHUNK = 128
_CT = 80
_EPT = _CT * _CHUNK
_EPAD = _NW * _EPT
_NACC = 10112
_ZROWS = _NACC // _NS
_OROWS = _NACC // _NS

_BN = 2000
_NBLK = _N // _BN


@functools.cache
def _make_sc_agg():
    mesh = plsc.VectorSubcoreMesh(core_axis_name="c", subcore_axis_name="s")

    @functools.partial(
        pl.kernel,
        mesh=mesh,
        out_type=jax.ShapeDtypeStruct((_NC, _NACC, _D), jnp.float32),
        scratch_types=[
            pltpu.VMEM((_EPT // 128, 128), jnp.int32),
            pltpu.VMEM((_CHUNK,), jnp.int32),
            pltpu.VMEM((_CHUNK,), jnp.int32),
            pltpu.VMEM((_CHUNK,), jnp.int32),
            pltpu.VMEM((_CHUNK,), jnp.int32),
            pltpu.VMEM((_CHUNK, _D), jnp.float32),
            pltpu.VMEM((_CHUNK, _D), jnp.float32),
            pltpu.VMEM_SHARED((_NACC, _D), jnp.float32),
            pltpu.SemaphoreType.DMA,
            pltpu.SemaphoreType.DMA,
        ],
    )
    def agg(feat_hbm, edge_hbm, zeros_hbm, out_hbm,
            pk_v, src0_v, src1_v, dst0_v, dst1_v, rows0_v, rows1_v,
            acc_sh, gsem, ssem):
        c = lax.axis_index("c")
        s = lax.axis_index("s")
        wid = c * _NS + s
        srcs = (src0_v, src1_v)
        dsts = (dst0_v, dst1_v)
        rows = (rows0_v, rows1_v)

        def unpack(chunk, buf):
            for q in range(_CHUNK // 128):
                for k in range(8):
                    p = pk_v[chunk * (_CHUNK // 128) + q, pl.ds(k * 16, 16)]
                    col = pl.ds(q * 128 + k * 16, 16)
                    srcs[buf][col] = lax.bitwise_and(p, 0xFFFF)
                    dsts[buf][col] = lax.shift_right_logical(p, 16)

        def gather(buf):
            return pltpu.make_async_copy(
                feat_hbm.at[srcs[buf]], rows[buf], gsem)

        def scatter(buf):
            return pltpu.make_async_copy(
                rows[buf], acc_sh.at[dsts[buf]], ssem)

        pltpu.sync_copy(zeros_hbm, acc_sh.at[pl.ds(s * _ZROWS, _ZROWS)])
        pltpu.sync_copy(edge_hbm.at[wid], pk_v)
        unpack(jnp.int32(0), 0)
        gather(0).start()
        plsc.subcore_barrier()

        def group(g, carry):
            for par in range(2):
                j = g * 2 + par
                @pl.when(j >= 1)
                def _wait_prev_scatter():
                    scatter(1 - par).wait()

                @pl.when(j + 1 < _CT)
                def _prep_next():
                    unpack(j + 1, 1 - par)
                    gather(1 - par).start()

                gather(par).wait()
                pltpu.async_copy(
                    rows[par], acc_sh.at[dsts[par]], ssem, add=True)
            return carry

        lax.fori_loop(0, _CT // 2, group, 0, unroll=False)
        scatter((_CT - 1) % 2).wait()

        plsc.subcore_barrier()
        pltpu.sync_copy(acc_sh.at[pl.ds(s * _OROWS, _OROWS)],
                        out_hbm.at[c, pl.ds(s * _OROWS, _OROWS)])

    return agg


def _sc_agg(feat, packed, zrows):
    return _make_sc_agg()(feat, packed, zrows)


def _mlp_body(x_ref, agg_ref, wa_ref, ba_ref, wb_ref, bb_ref, out_ref):
    h = x_ref[...] + agg_ref[0] + agg_ref[1]
    h = jnp.dot(h, wa_ref[...], preferred_element_type=jnp.float32) + ba_ref[...]
    h = jnp.maximum(h, 0.0)
    h = jnp.dot(h, wb_ref[...], preferred_element_type=jnp.float32) + bb_ref[...]
    out_ref[...] = jnp.maximum(h, 0.0)


def _tc_mlp1(x, agg, wa, ba, wb, bb):
    blk = lambda i: (i, 0)
    full = lambda i: (0, 0)
    return pl.pallas_call(
        _mlp_body,
        grid=(_NBLK,),
        in_specs=[
            pl.BlockSpec((_BN, _D), blk),
            pl.BlockSpec((_NC, _BN, _D), lambda i: (0, i, 0)),
            pl.BlockSpec((_D, _D), full),
            pl.BlockSpec((1, _D), full),
            pl.BlockSpec((_D, _D), full),
            pl.BlockSpec((1, _D), full),
        ],
        out_specs=pl.BlockSpec((_BN, _D), blk),
        out_shape=jax.ShapeDtypeStruct((_N, _D), jnp.float32),
    )(x, agg, wa, ba, wb, bb)


def _mlp_pool_body(batch_ref, x_ref, agg_ref, wa_ref, ba_ref, wb_ref,
                   bb_ref, out_ref, pool_ref):
    i = pl.program_id(0)
    h = x_ref[...] + agg_ref[0] + agg_ref[1]
    h = jnp.dot(h, wa_ref[...], preferred_element_type=jnp.float32) + ba_ref[...]
    h = jnp.maximum(h, 0.0)
    h = jnp.dot(h, wb_ref[...], preferred_element_type=jnp.float32) + bb_ref[...]
    out_ref[...] = h
    b = batch_ref[0, 0, :]
    onehot = (b[:, None] == lax.broadcasted_iota(jnp.int32, (_BN, _G), 1))
    contrib = lax.dot_general(onehot.astype(jnp.float32), h,
                              (((0,), (0,)), ((), ())),
                              preferred_element_type=jnp.float32)

    @pl.when(i == 0)
    def _init():
        pool_ref[...] = jnp.zeros_like(pool_ref)

    pool_ref[...] += contrib


def _tc_mlp2_pool(batch3, x, agg, wa, ba, wb, bb):
    blk = lambda i: (i, 0)
    full = lambda i: (0, 0)
    return pl.pallas_call(
        _mlp_pool_body,
        grid=(_NBLK,),
        in_specs=[
            pl.BlockSpec((1, 1, _BN), lambda i: (i, 0, 0)),
            pl.BlockSpec((_BN, _D), blk),
            pl.BlockSpec((_NC, _BN, _D), lambda i: (0, i, 0)),
            pl.BlockSpec((_D, _D), full),
            pl.BlockSpec((1, _D), full),
            pl.BlockSpec((_D, _D), full),
            pl.BlockSpec((1, _D), full),
        ],
        out_specs=[
            pl.BlockSpec((_BN, _D), blk),
            pl.BlockSpec((_G, _D), full),
        ],
        out_shape=[
            jax.ShapeDtypeStruct((_N, _D), jnp.float32),
            jax.ShapeDtypeStruct((_G, _D), jnp.float32),
        ],
    )(batch3, x, agg, wa, ba, wb, bb)


def kernel(x, edge_index, batch, W1a, b1a, W1b, b1b, W2a, b2a, W2b, b2b):
    src = edge_index[0]
    dst = edge_index[1]
    pad = _EPAD - _E
    pad_src = jnp.arange(pad, dtype=jnp.int32) % _N
    pad_dst = _N + (jnp.arange(pad, dtype=jnp.int32) % (_NACC - _N))
    srcp = jnp.concatenate([src, pad_src])
    dstp = jnp.concatenate([dst, pad_dst])
    packed = jnp.bitwise_or(srcp, jnp.left_shift(dstp, 16))
    packed = packed.reshape(_NW, _EPT // 128, 128)
    zrows = jnp.zeros((_ZROWS, _D), jnp.float32)

    ba1 = b1a.reshape(1, _D)
    bb1 = b1b.reshape(1, _D)
    ba2 = b2a.reshape(1, _D)
    bb2 = b2b.reshape(1, _D)

    agg1 = _sc_agg(x, packed, zrows)
    h1 = _tc_mlp1(x, agg1, W1a, ba1, W1b, bb1)
    agg2 = _sc_agg(h1, packed, zrows)
    batch3 = batch.reshape(_NBLK, 1, _BN)
    h2, pooled = _tc_mlp2_pool(batch3, h1, agg2, W2a, ba2, W2b, bb2)
    return (pooled, h2)


# speedup vs baseline: 4.0790x; 1.0067x over previous
import functools

import jax
import jax.numpy as jnp
from jax import lax
from jax.experimental import pallas as pl
from jax.experimental.pallas import tpu as pltpu
from jax.experimental.pallas import tpu_sc as plsc

_N = 10000
_E = 320000
_D = 128
_G = 64

_NC = 2
_NS = 16
_NW = _NC * _NS
_CHUNK = 128
_CT = 80
_EPT = _CT * _CHUNK
_EPAD = _NW * _EPT
_NACC = 10112
_ZROWS = _NACC // _NS
_OROWS = _NACC // _NS

_BN = 2000
_NBLK = _N // _BN


@functools.cache
def _make_sc_agg():
    mesh = plsc.VectorSubcoreMesh(core_axis_name="c", subcore_axis_name="s")

    @functools.partial(
        pl.kernel,
        mesh=mesh,
        out_type=jax.ShapeDtypeStruct((_NC, _NACC, _D), jnp.float32),
        scratch_types=[
            pltpu.VMEM((_EPT // 128, 128), jnp.int32),
            pltpu.VMEM((_CHUNK,), jnp.int32),
            pltpu.VMEM((_CHUNK,), jnp.int32),
            pltpu.VMEM((_CHUNK,), jnp.int32),
            pltpu.VMEM((_CHUNK,), jnp.int32),
            pltpu.VMEM((_CHUNK, _D), jnp.float32),
            pltpu.VMEM((_CHUNK, _D), jnp.float32),
            pltpu.VMEM_SHARED((_NACC, _D), jnp.float32),
            pltpu.SemaphoreType.DMA,
            pltpu.SemaphoreType.DMA,
        ],
    )
    def agg(feat_hbm, edge_hbm, zeros_hbm, out_hbm,
            pk_v, src0_v, src1_v, dst0_v, dst1_v, rows0_v, rows1_v,
            acc_sh, gsem, ssem):
        c = lax.axis_index("c")
        s = lax.axis_index("s")
        wid = c * _NS + s
        srcs = (src0_v, src1_v)
        dsts = (dst0_v, dst1_v)
        rows = (rows0_v, rows1_v)

        def unpack(chunk, buf):
            for q in range(_CHUNK // 128):
                for k in range(8):
                    p = pk_v[chunk * (_CHUNK // 128) + q, pl.ds(k * 16, 16)]
                    col = pl.ds(q * 128 + k * 16, 16)
                    srcs[buf][col] = lax.bitwise_and(p, 0xFFFF)
                    dsts[buf][col] = lax.shift_right_logical(p, 16)

        def gather(buf):
            return pltpu.make_async_copy(
                feat_hbm.at[srcs[buf]], rows[buf], gsem)

        def scatter(buf):
            return pltpu.make_async_copy(
                rows[buf], acc_sh.at[dsts[buf]], ssem)

        pltpu.sync_copy(zeros_hbm.at[pl.ds(s * _ZROWS, _ZROWS)],
                        acc_sh.at[pl.ds(s * _ZROWS, _ZROWS)])
        pltpu.sync_copy(edge_hbm.at[wid], pk_v)
        unpack(jnp.int32(0), 0)
        gather(0).start()
        plsc.subcore_barrier()

        def group(g, carry):
            for par in range(2):
                j = g * 2 + par
                @pl.when(j >= 1)
                def _wait_prev_scatter():
                    scatter(1 - par).wait()

                @pl.when(j + 1 < _CT)
                def _prep_next():
                    unpack(j + 1, 1 - par)
                    gather(1 - par).start()

                gather(par).wait()
                pltpu.async_copy(
                    rows[par], acc_sh.at[dsts[par]], ssem, add=True)
            return carry

        lax.fori_loop(0, _CT // 2, group, 0, unroll=False)
        scatter((_CT - 1) % 2).wait()

        plsc.subcore_barrier()
        pltpu.sync_copy(acc_sh.at[pl.ds(s * _OROWS, _OROWS)],
                        out_hbm.at[c, pl.ds(s * _OROWS, _OROWS)])

    return agg


def _sc_agg(feat, packed, zrows):
    return _make_sc_agg()(feat, packed, zrows)


def _mlp_body(x_ref, agg_ref, wa_ref, ba_ref, wb_ref, bb_ref, out_ref):
    h = x_ref[...] + agg_ref[0] + agg_ref[1]
    h = jnp.dot(h, wa_ref[...], preferred_element_type=jnp.float32) + ba_ref[...]
    h = jnp.maximum(h, 0.0)
    h = jnp.dot(h, wb_ref[...], preferred_element_type=jnp.float32) + bb_ref[...]
    out_ref[...] = jnp.maximum(h, 0.0)


def _tc_mlp1(x, agg, wa, ba, wb, bb):
    blk = lambda i: (i, 0)
    full = lambda i: (0, 0)
    return pl.pallas_call(
        _mlp_body,
        grid=(_NBLK,),
        in_specs=[
            pl.BlockSpec((_BN, _D), blk),
            pl.BlockSpec((_NC, _BN, _D), lambda i: (0, i, 0)),
            pl.BlockSpec((_D, _D), full),
            pl.BlockSpec((1, _D), full),
            pl.BlockSpec((_D, _D), full),
            pl.BlockSpec((1, _D), full),
        ],
        out_specs=pl.BlockSpec((_BN, _D), blk),
        out_shape=jax.ShapeDtypeStruct((_N, _D), jnp.float32),
    )(x, agg, wa, ba, wb, bb)


def _mlp_pool_body(batch_ref, x_ref, agg_ref, wa_ref, ba_ref, wb_ref,
                   bb_ref, out_ref, pool_ref):
    i = pl.program_id(0)
    h = x_ref[...] + agg_ref[0] + agg_ref[1]
    h = jnp.dot(h, wa_ref[...], preferred_element_type=jnp.float32) + ba_ref[...]
    h = jnp.maximum(h, 0.0)
    h = jnp.dot(h, wb_ref[...], preferred_element_type=jnp.float32) + bb_ref[...]
    out_ref[...] = h
    b = batch_ref[0, 0, :]
    onehot = (b[:, None] == lax.broadcasted_iota(jnp.int32, (_BN, _G), 1))
    contrib = lax.dot_general(onehot.astype(jnp.float32), h,
                              (((0,), (0,)), ((), ())),
                              preferred_element_type=jnp.float32)

    @pl.when(i == 0)
    def _init():
        pool_ref[...] = jnp.zeros_like(pool_ref)

    pool_ref[...] += contrib


def _tc_mlp2_pool(batch3, x, agg, wa, ba, wb, bb):
    blk = lambda i: (i, 0)
    full = lambda i: (0, 0)
    return pl.pallas_call(
        _mlp_pool_body,
        grid=(_NBLK,),
        in_specs=[
            pl.BlockSpec((1, 1, _BN), lambda i: (i, 0, 0)),
            pl.BlockSpec((_BN, _D), blk),
            pl.BlockSpec((_NC, _BN, _D), lambda i: (0, i, 0)),
            pl.BlockSpec((_D, _D), full),
            pl.BlockSpec((1, _D), full),
            pl.BlockSpec((_D, _D), full),
            pl.BlockSpec((1, _D), full),
        ],
        out_specs=[
            pl.BlockSpec((_BN, _D), blk),
            pl.BlockSpec((_G, _D), full),
        ],
        out_shape=[
            jax.ShapeDtypeStruct((_N, _D), jnp.float32),
            jax.ShapeDtypeStruct((_G, _D), jnp.float32),
        ],
    )(batch3, x, agg, wa, ba, wb, bb)


def kernel(x, edge_index, batch, W1a, b1a, W1b, b1b, W2a, b2a, W2b, b2b):
    src = edge_index[0]
    dst = edge_index[1]
    pad = _EPAD - _E
    pad_src = jnp.arange(pad, dtype=jnp.int32) % _N
    pad_dst = _N + (jnp.arange(pad, dtype=jnp.int32) % (_NACC - _N))
    srcp = jnp.concatenate([src, pad_src])
    dstp = jnp.concatenate([dst, pad_dst])
    packed = jnp.bitwise_or(srcp, jnp.left_shift(dstp, 16))
    packed = packed.reshape(_NW, _EPT // 128, 128)
    zrows = jnp.zeros((_NACC, _D), jnp.float32)

    ba1 = b1a.reshape(1, _D)
    bb1 = b1b.reshape(1, _D)
    ba2 = b2a.reshape(1, _D)
    bb2 = b2b.reshape(1, _D)

    agg1 = _sc_agg(x, packed, zrows)
    h1 = _tc_mlp1(x, agg1, W1a, ba1, W1b, bb1)
    agg2 = _sc_agg(h1, packed, zrows)
    batch3 = batch.reshape(_NBLK, 1, _BN)
    h2, pooled = _tc_mlp2_pool(batch3, h1, agg2, W2a, ba2, W2b, bb2)
    return (pooled, h2)
